# baseline (speedup 1.0000x reference)
import functools

import jax
import jax.numpy as jnp
from jax import lax
from jax.experimental import pallas as pl
from jax.experimental.pallas import tpu as pltpu
from jax.experimental.pallas import tpu_sc as plsc

N = 10000
E = 320000
D = 128
H = 128
C = 16
G = 64

NC = 2
NS = 16
NW = NC * NS
EPW = E // NW
K = 16
NCH = EPW // K
NBUF = 5
SLK = 1
CCH = 2000
RPT = 624
RLAST = N - (NS - 1) * RPT


def _make_agg_body(with_counts):
    def _agg_body(x_hbm, src_hbm, dst_hbm, zrow_hbm, out_hbm, outc_hbm,
                  acc, cnt, sidx, didx, rows, ones, cntv, gsems, ssems,
                  csem):
        c = lax.axis_index("c")
        s = lax.axis_index("s")
        wid = s * NC + c

        pltpu.sync_copy(src_hbm.at[wid], sidx)
        pltpu.sync_copy(dst_hbm.at[wid], didx)

        def sl(ref, g):
            return ref.at[pl.ds(g * K, K)]

        if with_counts:
            for j in range(-(-K // 16)):
                ones[pl.ds(j * 16, 16)] = jnp.ones((16,), jnp.float32)

        @pl.when(s < NS - 1)
        def _():
            pltpu.sync_copy(zrow_hbm.at[pl.ds(s * RPT, RPT)],
                            acc.at[pl.ds(s * RPT, RPT)])

        @pl.when(s == NS - 1)
        def _():
            pltpu.sync_copy(zrow_hbm.at[pl.ds((NS - 1) * RPT, RLAST)],
                            acc.at[pl.ds((NS - 1) * RPT, RLAST)])

        if with_counts:
            @pl.when(s == 0)
            def _():
                def zb(j, _):
                    cntv[pl.ds(j * 16, 16)] = jnp.zeros((16,), jnp.float32)
                    return 0
                lax.fori_loop(0, CCH // 16, zb, 0)
                def zc(j, _):
                    pltpu.sync_copy(cntv, cnt.at[pl.ds(j * CCH, CCH)])
                    return 0
                lax.fori_loop(0, N // CCH, zc, 0)

        plsc.subcore_barrier()

        def gather(g, b):
            pltpu.async_copy(x_hbm.at[sl(sidx, g)], rows.at[b], gsems.at[b])

        def gather_wait(g, b):
            pltpu.make_async_copy(x_hbm.at[sl(sidx, g)], rows.at[b],
                                  gsems.at[b]).wait()

        def scatter(g, b):
            pltpu.async_copy(rows.at[b], acc.at[sl(didx, g)], ssems.at[b],
                             add=True)

        def scatter_wait(g, b):
            pltpu.make_async_copy(rows.at[b], acc.at[sl(didx, g)],
                                  ssems.at[b]).wait()

        for b in range(NBUF - SLK):
            gather(b, b)

        def body(i, _):
            for b in range(NBUF):
                g = i * NBUF + b
                gather_wait(g, b)
                scatter(g, b)
                if with_counts:
                    @pl.when(g > 0)
                    def _():
                        pltpu.make_async_copy(ones.at[pl.ds(0, K)],
                                              cnt.at[sl(didx, g)],
                                              csem).wait()
                    pltpu.async_copy(ones.at[pl.ds(0, K)], cnt.at[sl(didx, g)],
                                     csem, add=True)
                @pl.when(g >= SLK)
                def _():
                    scatter_wait(g - SLK, (b - SLK) % NBUF)

                @pl.when(g + NBUF - SLK < NCH)
                def _():
                    gather(g + NBUF - SLK, (b + NBUF - SLK) % NBUF)
            return 0

        lax.fori_loop(0, NCH // NBUF, body, 0)

        for q in range(SLK):
            g = NCH - SLK + q
            scatter_wait(g, g % NBUF)
        if with_counts:
            pltpu.make_async_copy(ones.at[pl.ds(0, K)], cnt.at[sl(didx, 0)],
                                  csem).wait()

        plsc.subcore_barrier()

        @pl.when(s < NS - 1)
        def _():
            pltpu.sync_copy(acc.at[pl.ds(s * RPT, RPT)],
                            out_hbm.at[c, pl.ds(s * RPT, RPT)])

        @pl.when(s == NS - 1)
        def _():
            pltpu.sync_copy(acc.at[pl.ds((NS - 1) * RPT, RLAST)],
                            out_hbm.at[c, pl.ds((NS - 1) * RPT, RLAST)])

        if with_counts:
            @pl.when(s == 0)
            def _():
                def cc(j, _):
                    pltpu.sync_copy(cnt.at[pl.ds(j * CCH, CCH)], cntv)
                    pltpu.sync_copy(
                        cntv, outc_hbm.at[pl.ds(c * N + j * CCH, CCH)])
                    return 0
                lax.fori_loop(0, N // CCH, cc, 0)

    return _agg_body


def _make_agg_call(with_counts):
    return functools.partial(
        pl.kernel,
        out_type=(jax.ShapeDtypeStruct((NC, N, D), jnp.float32),
                  jax.ShapeDtypeStruct((NC * N,), jnp.float32)),
        mesh=plsc.VectorSubcoreMesh(core_axis_name="c",
                                    subcore_axis_name="s"),
        scratch_types=[
            pltpu.VMEM_SHARED((N, D), jnp.float32),
            pltpu.VMEM_SHARED((N,), jnp.float32),
            pltpu.VMEM((EPW,), jnp.int32),
            pltpu.VMEM((EPW,), jnp.int32),
            pltpu.VMEM((NBUF, K, D), jnp.float32),
            pltpu.VMEM((16 * -(-K // 16),), jnp.float32),
            pltpu.VMEM((CCH,), jnp.float32),
            pltpu.SemaphoreType.DMA((NBUF,)),
            pltpu.SemaphoreType.DMA((NBUF,)),
            pltpu.SemaphoreType.DMA,
        ],
    )(_make_agg_body(with_counts))


_agg_counts = _make_agg_call(True)
_agg_plain = _make_agg_call(False)


R = 1000
NB = N // R


def _dense1_body(pp, cp, x, wl, wr, b, out):
    inv = 1.0 / jnp.maximum(cp[0] + cp[1], 1.0)
    agg = (pp[0] + pp[1]) * inv
    h = (jnp.dot(agg, wl[...], preferred_element_type=jnp.float32)
         + jnp.dot(x[...], wr[...], preferred_element_type=jnp.float32)
         + b[...])
    out[...] = jnp.maximum(h, 0.0)


def _dense1(Pp, Cp, x, wl_t, wr_t, b):
    return pl.pallas_call(
        _dense1_body,
        grid=(NB,),
        in_specs=[
            pl.BlockSpec((NC, R, D), lambda i: (0, i, 0)),
            pl.BlockSpec((NC, R, 1), lambda i: (0, i, 0)),
            pl.BlockSpec((R, D), lambda i: (i, 0)),
            pl.BlockSpec((D, H), lambda i: (0, 0)),
            pl.BlockSpec((D, H), lambda i: (0, 0)),
            pl.BlockSpec((1, H), lambda i: (0, 0)),
        ],
        out_specs=pl.BlockSpec((R, H), lambda i: (i, 0)),
        out_shape=jax.ShapeDtypeStruct((N, H), jnp.float32),
    )(Pp, Cp, x, wl_t, wr_t, b)


def _dense2_body(pp, cp, h1, bidx, wl, wr, b, wfc, bfc, out,
                 pooled, gcnt):
    i = pl.program_id(0)

    inv = 1.0 / jnp.maximum(cp[0] + cp[1], 1.0)
    agg = (pp[0] + pp[1]) * inv
    h2 = (jnp.dot(agg, wl[...], preferred_element_type=jnp.float32)
          + jnp.dot(h1[...], wr[...], preferred_element_type=jnp.float32)
          + b[...])
    h2 = jnp.maximum(h2, 0.0)

    bids = bidx[0]
    gids = lax.broadcasted_iota(jnp.int32, (G, R), 0)
    oh = jnp.where(gids == jnp.broadcast_to(bids, (G, R)), 1.0, 0.0)

    @pl.when(i == 0)
    def _():
        pooled[...] = jnp.zeros_like(pooled)
        gcnt[...] = jnp.zeros_like(gcnt)

    pooled[...] += jnp.dot(oh, h2, preferred_element_type=jnp.float32)
    gcnt[...] += jnp.sum(oh, axis=1, keepdims=True)

    @pl.when(i == NB - 1)
    def _():
        mean = pooled[...] / jnp.maximum(gcnt[...], 1.0)
        logits = (jnp.dot(mean, wfc[...], preferred_element_type=jnp.float32)
                  + bfc[...])
        z = logits - jnp.max(logits, axis=-1, keepdims=True)
        e = jnp.exp(z)
        out[...] = e / jnp.sum(e, axis=-1, keepdims=True)


def _dense2(Pp, Cp, h1, batch3, wl_t, wr_t, b, wfc_t, bfc):
    return pl.pallas_call(
        _dense2_body,
        grid=(NB,),
        in_specs=[
            pl.BlockSpec((NC, R, D), lambda i: (0, i, 0)),
            pl.BlockSpec((NC, R, 1), lambda i: (0, i, 0)),
            pl.BlockSpec((R, D), lambda i: (i, 0)),
            pl.BlockSpec((1, 1, R), lambda i: (i, 0, 0)),
            pl.BlockSpec((D, H), lambda i: (0, 0)),
            pl.BlockSpec((D, H), lambda i: (0, 0)),
            pl.BlockSpec((1, H), lambda i: (0, 0)),
            pl.BlockSpec((H, C), lambda i: (0, 0)),
            pl.BlockSpec((1, C), lambda i: (0, 0)),
        ],
        out_specs=pl.BlockSpec((G, C), lambda i: (0, 0)),
        out_shape=jax.ShapeDtypeStruct((G, C), jnp.float32),
        scratch_shapes=[
            pltpu.VMEM((G, H), jnp.float32),
            pltpu.VMEM((G, 1), jnp.float32),
        ],
    )(Pp, Cp, h1, batch3, wl_t, wr_t, b, wfc_t, bfc)


def kernel(x, edge_index, batch, W1l, b1, W1r, W2l, b2, W2r, Wfc, bfc):
    src = edge_index[0].reshape(NW, EPW)
    dst = edge_index[1].reshape(NW, EPW)
    zrow = jnp.zeros((N, D), jnp.float32)

    P1, C1 = _agg_counts(x, src, dst, zrow)
    h1 = _dense1(P1, C1.reshape(NC, N, 1), x,
                 W1l.T, W1r.T, b1.reshape(1, H))

    P2, _ = _agg_plain(h1, src, dst, zrow)
    out = _dense2(P2, C1.reshape(NC, N, 1), h1, batch.reshape(NB, 1, R),
                  W2l.T, W2r.T, b2.reshape(1, H),
                  Wfc.T, bfc.reshape(1, C))
    return out


# speedup vs baseline: 1.2776x; 1.2776x over previous
import functools

import jax
import jax.numpy as jnp
from jax import lax
from jax.experimental import pallas as pl
from jax.experimental.pallas import tpu as pltpu
from jax.experimental.pallas import tpu_sc as plsc

N = 10000
E = 320000
D = 128
H = 128
C = 16
G = 64

NC = 2
NS = 16
NW = NC * NS
EPW = E // NW
K = 80
NCH = EPW // K
NBUF = 3
REM = NCH - (NCH // NBUF) * NBUF
SLK = 1
CCH = 2000
RPT = 624
RLAST = N - (NS - 1) * RPT


def _make_agg_body(with_counts):
    def _agg_body(x_hbm, pidx_hbm, zrow_hbm, out_hbm, outc_hbm,
                  acc, cnt, pidx, sidxc, didxc, rows, ones, cntv, gsems,
                  ssems, csem):
        c = lax.axis_index("c")
        s = lax.axis_index("s")
        wid = s * NC + c

        pltpu.sync_copy(pidx_hbm.at[wid], pidx)

        if with_counts:
            for j in range(-(-K // 16)):
                ones[pl.ds(j * 16, 16)] = jnp.ones((16,), jnp.float32)

        @pl.when(s < NS - 1)
        def _():
            pltpu.sync_copy(zrow_hbm.at[pl.ds(s * RPT, RPT)],
                            acc.at[pl.ds(s * RPT, RPT)])

        @pl.when(s == NS - 1)
        def _():
            pltpu.sync_copy(zrow_hbm.at[pl.ds((NS - 1) * RPT, RLAST)],
                            acc.at[pl.ds((NS - 1) * RPT, RLAST)])

        if with_counts:
            @pl.when(s == 0)
            def _():
                def zb(j, _):
                    cntv[pl.ds(j * 16, 16)] = jnp.zeros((16,), jnp.float32)
                    return 0
                lax.fori_loop(0, CCH // 16, zb, 0)
                def zc(j, _):
                    pltpu.sync_copy(cntv, cnt.at[pl.ds(j * CCH, CCH)])
                    return 0
                lax.fori_loop(0, N // CCH, zc, 0)

        plsc.subcore_barrier()

        def gather(g, b):
            for j in range(K // 16):
                v = pidx[pl.ds(g * K + j * 16, 16)]
                sidxc[b, pl.ds(j * 16, 16)] = lax.shift_right_logical(v, 14)
                didxc[b, pl.ds(j * 16, 16)] = v & 16383
            pltpu.async_copy(x_hbm.at[sidxc.at[b]], rows.at[b], gsems.at[b])

        def gather_wait(b):
            pltpu.make_async_copy(x_hbm.at[sidxc.at[b]], rows.at[b],
                                  gsems.at[b]).wait()

        def scatter(b):
            pltpu.async_copy(rows.at[b], acc.at[didxc.at[b]], ssems.at[b],
                             add=True)

        def scatter_wait(b):
            pltpu.make_async_copy(rows.at[b], acc.at[didxc.at[b]],
                                  ssems.at[b]).wait()

        def count_scatter(b, first):
            if first:
                pltpu.async_copy(ones.at[pl.ds(0, K)], cnt.at[didxc.at[b]],
                                 csem, add=True)
            else:
                pltpu.make_async_copy(ones.at[pl.ds(0, K)],
                                      cnt.at[didxc.at[b]], csem).wait()
                pltpu.async_copy(ones.at[pl.ds(0, K)], cnt.at[didxc.at[b]],
                                 csem, add=True)

        for b in range(NBUF - SLK):
            gather(b, b)

        def body(i, _):
            for b in range(NBUF):
                g = i * NBUF + b
                gather_wait(b)
                scatter(b)
                if with_counts:
                    @pl.when(g > 0)
                    def _():
                        count_scatter(b, False)

                    @pl.when(g == 0)
                    def _():
                        count_scatter(b, True)
                @pl.when(g >= SLK)
                def _():
                    scatter_wait((b - SLK) % NBUF)

                @pl.when(g + NBUF - SLK < NCH)
                def _():
                    gather(g + NBUF - SLK, (b + NBUF - SLK) % NBUF)
            return 0

        lax.fori_loop(0, NCH // NBUF, body, 0)

        for q in range(REM):
            g = (NCH // NBUF) * NBUF + q
            b = g % NBUF
            gather_wait(b)
            scatter(b)
            if with_counts:
                count_scatter(b, False)
            scatter_wait((b - SLK) % NBUF)

        for q in range(SLK):
            g = NCH - SLK + q
            scatter_wait(g % NBUF)
        if with_counts:
            pltpu.make_async_copy(ones.at[pl.ds(0, K)], cnt.at[didxc.at[0]],
                                  csem).wait()

        plsc.subcore_barrier()

        @pl.when(s < NS - 1)
        def _():
            pltpu.sync_copy(acc.at[pl.ds(s * RPT, RPT)],
                            out_hbm.at[c, pl.ds(s * RPT, RPT)])

        @pl.when(s == NS - 1)
        def _():
            pltpu.sync_copy(acc.at[pl.ds((NS - 1) * RPT, RLAST)],
                            out_hbm.at[c, pl.ds((NS - 1) * RPT, RLAST)])

        if with_counts:
            @pl.when(s == 0)
            def _():
                def cc(j, _):
                    pltpu.sync_copy(cnt.at[pl.ds(j * CCH, CCH)], cntv)
                    pltpu.sync_copy(
                        cntv, outc_hbm.at[pl.ds(c * N + j * CCH, CCH)])
                    return 0
                lax.fori_loop(0, N // CCH, cc, 0)

    return _agg_body


def _make_agg_call(with_counts):
    return functools.partial(
        pl.kernel,
        out_type=(jax.ShapeDtypeStruct((NC, N, D), jnp.float32),
                  jax.ShapeDtypeStruct((NC * N,), jnp.float32)),
        mesh=plsc.VectorSubcoreMesh(core_axis_name="c",
                                    subcore_axis_name="s"),
        scratch_types=[
            pltpu.VMEM_SHARED((N, D), jnp.float32),
            pltpu.VMEM_SHARED((N,), jnp.float32),
            pltpu.VMEM((EPW,), jnp.int32),
            pltpu.VMEM((NBUF, K), jnp.int32),
            pltpu.VMEM((NBUF, K), jnp.int32),
            pltpu.VMEM((NBUF, K, D), jnp.float32),
            pltpu.VMEM((16 * -(-K // 16),), jnp.float32),
            pltpu.VMEM((CCH,), jnp.float32),
            pltpu.SemaphoreType.DMA((NBUF,)),
            pltpu.SemaphoreType.DMA((NBUF,)),
            pltpu.SemaphoreType.DMA,
        ],
    )(_make_agg_body(with_counts))


_agg_counts = _make_agg_call(True)
_agg_plain = _make_agg_call(False)


R = 1000
NB = N // R


def _dense1_body(pp, cp, x, wl, wr, b, out):
    inv = 1.0 / jnp.maximum(cp[0] + cp[1], 1.0)
    agg = (pp[0] + pp[1]) * inv
    h = (jnp.dot(agg, wl[...], preferred_element_type=jnp.float32)
         + jnp.dot(x[...], wr[...], preferred_element_type=jnp.float32)
         + b[...])
    out[...] = jnp.maximum(h, 0.0)


def _dense1(Pp, Cp, x, wl_t, wr_t, b):
    return pl.pallas_call(
        _dense1_body,
        grid=(NB,),
        in_specs=[
            pl.BlockSpec((NC, R, D), lambda i: (0, i, 0)),
            pl.BlockSpec((NC, R, 1), lambda i: (0, i, 0)),
            pl.BlockSpec((R, D), lambda i: (i, 0)),
            pl.BlockSpec((D, H), lambda i: (0, 0)),
            pl.BlockSpec((D, H), lambda i: (0, 0)),
            pl.BlockSpec((1, H), lambda i: (0, 0)),
        ],
        out_specs=pl.BlockSpec((R, H), lambda i: (i, 0)),
        out_shape=jax.ShapeDtypeStruct((N, H), jnp.float32),
    )(Pp, Cp, x, wl_t, wr_t, b)


def _dense2_body(pp, cp, h1, bidx, wl, wr, b, wfc, bfc, out,
                 pooled, gcnt):
    i = pl.program_id(0)

    inv = 1.0 / jnp.maximum(cp[0] + cp[1], 1.0)
    agg = (pp[0] + pp[1]) * inv
    h2 = (jnp.dot(agg, wl[...], preferred_element_type=jnp.float32)
          + jnp.dot(h1[...], wr[...], preferred_element_type=jnp.float32)
          + b[...])
    h2 = jnp.maximum(h2, 0.0)

    bids = bidx[0]
    gids = lax.broadcasted_iota(jnp.int32, (G, R), 0)
    oh = jnp.where(gids == jnp.broadcast_to(bids, (G, R)), 1.0, 0.0)

    @pl.when(i == 0)
    def _():
        pooled[...] = jnp.zeros_like(pooled)
        gcnt[...] = jnp.zeros_like(gcnt)

    pooled[...] += jnp.dot(oh, h2, preferred_element_type=jnp.float32)
    gcnt[...] += jnp.sum(oh, axis=1, keepdims=True)

    @pl.when(i == NB - 1)
    def _():
        mean = pooled[...] / jnp.maximum(gcnt[...], 1.0)
        logits = (jnp.dot(mean, wfc[...], preferred_element_type=jnp.float32)
                  + bfc[...])
        z = logits - jnp.max(logits, axis=-1, keepdims=True)
        e = jnp.exp(z)
        out[...] = e / jnp.sum(e, axis=-1, keepdims=True)


def _dense2(Pp, Cp, h1, batch3, wl_t, wr_t, b, wfc_t, bfc):
    return pl.pallas_call(
        _dense2_body,
        grid=(NB,),
        in_specs=[
            pl.BlockSpec((NC, R, D), lambda i: (0, i, 0)),
            pl.BlockSpec((NC, R, 1), lambda i: (0, i, 0)),
            pl.BlockSpec((R, D), lambda i: (i, 0)),
            pl.BlockSpec((1, 1, R), lambda i: (i, 0, 0)),
            pl.BlockSpec((D, H), lambda i: (0, 0)),
            pl.BlockSpec((D, H), lambda i: (0, 0)),
            pl.BlockSpec((1, H), lambda i: (0, 0)),
            pl.BlockSpec((H, C), lambda i: (0, 0)),
            pl.BlockSpec((1, C), lambda i: (0, 0)),
        ],
        out_specs=pl.BlockSpec((G, C), lambda i: (0, 0)),
        out_shape=jax.ShapeDtypeStruct((G, C), jnp.float32),
        scratch_shapes=[
            pltpu.VMEM((G, H), jnp.float32),
            pltpu.VMEM((G, 1), jnp.float32),
        ],
    )(Pp, Cp, h1, batch3, wl_t, wr_t, b, wfc_t, bfc)


def kernel(x, edge_index, batch, W1l, b1, W1r, W2l, b2, W2r, Wfc, bfc):
    src = edge_index[0].astype(jnp.int32)
    dst = edge_index[1].astype(jnp.int32)
    pidx = ((src << 14) | dst).reshape(NW, EPW)
    zrow = jnp.zeros((N, D), jnp.float32)

    P1, C1 = _agg_counts(x, pidx, zrow)
    h1 = _dense1(P1, C1.reshape(NC, N, 1), x,
                 W1l.T, W1r.T, b1.reshape(1, H))

    P2, _ = _agg_plain(h1, pidx, zrow)
    out = _dense2(P2, C1.reshape(NC, N, 1), h1, batch.reshape(NB, 1, R),
                  W2l.T, W2r.T, b2.reshape(1, H),
                  Wfc.T, bfc.reshape(1, C))
    return out


# speedup vs baseline: 1.2978x; 1.0158x over previous
import functools

import jax
import jax.numpy as jnp
from jax import lax
from jax.experimental import pallas as pl
from jax.experimental.pallas import tpu as pltpu
from jax.experimental.pallas import tpu_sc as plsc

N = 10000
E = 320000
D = 128
H = 128
C = 16
G = 64

NC = 2
NS = 16
NW = NC * NS
EPW = E // NW
K = 40
NCH = EPW // K
NBUF = 5
SLK = 1
CCH = 2000
RPT = 624
RLAST = N - (NS - 1) * RPT


def _make_agg_body(with_counts):
    def _agg_body(x_hbm, src_hbm, dst_hbm, zrow_hbm, out_hbm, outc_hbm,
                  acc, cnt, sidx, didx, rows, ones, cntv, gsems, ssems,
                  csem):
        c = lax.axis_index("c")
        s = lax.axis_index("s")
        wid = s * NC + c

        pltpu.sync_copy(src_hbm.at[wid], sidx)
        pltpu.sync_copy(dst_hbm.at[wid], didx)

        def sl(ref, g):
            return ref.at[pl.ds(g * K, K)]

        if with_counts:
            for j in range(-(-K // 16)):
                ones[pl.ds(j * 16, 16)] = jnp.ones((16,), jnp.float32)

        @pl.when(s < NS - 1)
        def _():
            pltpu.sync_copy(zrow_hbm.at[pl.ds(s * RPT, RPT)],
                            acc.at[pl.ds(s * RPT, RPT)])

        @pl.when(s == NS - 1)
        def _():
            pltpu.sync_copy(zrow_hbm.at[pl.ds((NS - 1) * RPT, RLAST)],
                            acc.at[pl.ds((NS - 1) * RPT, RLAST)])

        if with_counts:
            @pl.when(s == 0)
            def _():
                def zb(j, _):
                    cntv[pl.ds(j * 16, 16)] = jnp.zeros((16,), jnp.float32)
                    return 0
                lax.fori_loop(0, CCH // 16, zb, 0)
                def zc(j, _):
                    pltpu.sync_copy(cntv, cnt.at[pl.ds(j * CCH, CCH)])
                    return 0
                lax.fori_loop(0, N // CCH, zc, 0)

        plsc.subcore_barrier()

        def gather(g, b):
            pltpu.async_copy(x_hbm.at[sl(sidx, g)], rows.at[b], gsems.at[b])

        def gather_wait(g, b):
            pltpu.make_async_copy(x_hbm.at[sl(sidx, g)], rows.at[b],
                                  gsems.at[b]).wait()

        def scatter(g, b):
            pltpu.async_copy(rows.at[b], acc.at[sl(didx, g)], ssems.at[b],
                             add=True)

        def scatter_wait(g, b):
            pltpu.make_async_copy(rows.at[b], acc.at[sl(didx, g)],
                                  ssems.at[b]).wait()

        for b in range(NBUF - SLK):
            gather(b, b)

        def body(i, _):
            for b in range(NBUF):
                g = i * NBUF + b
                gather_wait(g, b)
                scatter(g, b)
                if with_counts:
                    @pl.when(g >= SLK)
                    def _():
                        pltpu.make_async_copy(
                            ones.at[pl.ds(0, K)], cnt.at[sl(didx, g)],
                            csem.at[(b - SLK) % NBUF]).wait()
                    pltpu.async_copy(ones.at[pl.ds(0, K)], cnt.at[sl(didx, g)],
                                     csem.at[b], add=True)
                @pl.when(g >= SLK)
                def _():
                    scatter_wait(g - SLK, (b - SLK) % NBUF)

                @pl.when(g + NBUF - SLK < NCH)
                def _():
                    gather(g + NBUF - SLK, (b + NBUF - SLK) % NBUF)
            return 0

        lax.fori_loop(0, NCH // NBUF, body, 0)

        for q in range(SLK):
            g = NCH - SLK + q
            scatter_wait(g, g % NBUF)
        if with_counts:
            for q in range(SLK):
                g = NCH - SLK + q
                pltpu.make_async_copy(ones.at[pl.ds(0, K)],
                                      cnt.at[sl(didx, 0)],
                                      csem.at[g % NBUF]).wait()

        plsc.subcore_barrier()

        @pl.when(s < NS - 1)
        def _():
            pltpu.sync_copy(acc.at[pl.ds(s * RPT, RPT)],
                            out_hbm.at[c, pl.ds(s * RPT, RPT)])

        @pl.when(s == NS - 1)
        def _():
            pltpu.sync_copy(acc.at[pl.ds((NS - 1) * RPT, RLAST)],
                            out_hbm.at[c, pl.ds((NS - 1) * RPT, RLAST)])

        if with_counts:
            @pl.when(s == 0)
            def _():
                def cc(j, _):
                    pltpu.sync_copy(cnt.at[pl.ds(j * CCH, CCH)], cntv)
                    pltpu.sync_copy(
                        cntv, outc_hbm.at[pl.ds(c * N + j * CCH, CCH)])
                    return 0
                lax.fori_loop(0, N // CCH, cc, 0)

    return _agg_body


def _make_agg_call(with_counts):
    return functools.partial(
        pl.kernel,
        out_type=(jax.ShapeDtypeStruct((NC, N, D), jnp.float32),
                  jax.ShapeDtypeStruct((NC * N,), jnp.float32)),
        mesh=plsc.VectorSubcoreMesh(core_axis_name="c",
                                    subcore_axis_name="s"),
        scratch_types=[
            pltpu.VMEM_SHARED((N, D), jnp.float32),
            pltpu.VMEM_SHARED((N,), jnp.float32),
            pltpu.VMEM((EPW,), jnp.int32),
            pltpu.VMEM((EPW,), jnp.int32),
            pltpu.VMEM((NBUF, K, D), jnp.float32),
            pltpu.VMEM((16 * -(-K // 16),), jnp.float32),
            pltpu.VMEM((CCH,), jnp.float32),
            pltpu.SemaphoreType.DMA((NBUF,)),
            pltpu.SemaphoreType.DMA((NBUF,)),
            pltpu.SemaphoreType.DMA((NBUF,)),
        ],
    )(_make_agg_body(with_counts))


_agg_counts = _make_agg_call(True)
_agg_plain = _make_agg_call(False)


R = 1000
NB = N // R


def _dense1_body(pp, cp, x, wl, wr, b, out):
    inv = 1.0 / jnp.maximum(cp[0] + cp[1], 1.0)
    agg = (pp[0] + pp[1]) * inv
    h = (jnp.dot(agg, wl[...], preferred_element_type=jnp.float32)
         + jnp.dot(x[...], wr[...], preferred_element_type=jnp.float32)
         + b[...])
    out[...] = jnp.maximum(h, 0.0)


def _dense1(Pp, Cp, x, wl_t, wr_t, b):
    return pl.pallas_call(
        _dense1_body,
        grid=(NB,),
        in_specs=[
            pl.BlockSpec((NC, R, D), lambda i: (0, i, 0)),
            pl.BlockSpec((NC, R, 1), lambda i: (0, i, 0)),
            pl.BlockSpec((R, D), lambda i: (i, 0)),
            pl.BlockSpec((D, H), lambda i: (0, 0)),
            pl.BlockSpec((D, H), lambda i: (0, 0)),
            pl.BlockSpec((1, H), lambda i: (0, 0)),
        ],
        out_specs=pl.BlockSpec((R, H), lambda i: (i, 0)),
        out_shape=jax.ShapeDtypeStruct((N, H), jnp.float32),
    )(Pp, Cp, x, wl_t, wr_t, b)


def _dense2_body(pp, cp, h1, bidx, wl, wr, b, wfc, bfc, out,
                 pooled, gcnt):
    i = pl.program_id(0)

    inv = 1.0 / jnp.maximum(cp[0] + cp[1], 1.0)
    agg = (pp[0] + pp[1]) * inv
    h2 = (jnp.dot(agg, wl[...], preferred_element_type=jnp.float32)
          + jnp.dot(h1[...], wr[...], preferred_element_type=jnp.float32)
          + b[...])
    h2 = jnp.maximum(h2, 0.0)

    bids = bidx[0]
    gids = lax.broadcasted_iota(jnp.int32, (G, R), 0)
    oh = jnp.where(gids == jnp.broadcast_to(bids, (G, R)), 1.0, 0.0)

    @pl.when(i == 0)
    def _():
        pooled[...] = jnp.zeros_like(pooled)
        gcnt[...] = jnp.zeros_like(gcnt)

    pooled[...] += jnp.dot(oh, h2, preferred_element_type=jnp.float32)
    gcnt[...] += jnp.sum(oh, axis=1, keepdims=True)

    @pl.when(i == NB - 1)
    def _():
        mean = pooled[...] / jnp.maximum(gcnt[...], 1.0)
        logits = (jnp.dot(mean, wfc[...], preferred_element_type=jnp.float32)
                  + bfc[...])
        z = logits - jnp.max(logits, axis=-1, keepdims=True)
        e = jnp.exp(z)
        out[...] = e / jnp.sum(e, axis=-1, keepdims=True)


def _dense2(Pp, Cp, h1, batch3, wl_t, wr_t, b, wfc_t, bfc):
    return pl.pallas_call(
        _dense2_body,
        grid=(NB,),
        in_specs=[
            pl.BlockSpec((NC, R, D), lambda i: (0, i, 0)),
            pl.BlockSpec((NC, R, 1), lambda i: (0, i, 0)),
            pl.BlockSpec((R, D), lambda i: (i, 0)),
            pl.BlockSpec((1, 1, R), lambda i: (i, 0, 0)),
            pl.BlockSpec((D, H), lambda i: (0, 0)),
            pl.BlockSpec((D, H), lambda i: (0, 0)),
            pl.BlockSpec((1, H), lambda i: (0, 0)),
            pl.BlockSpec((H, C), lambda i: (0, 0)),
            pl.BlockSpec((1, C), lambda i: (0, 0)),
        ],
        out_specs=pl.BlockSpec((G, C), lambda i: (0, 0)),
        out_shape=jax.ShapeDtypeStruct((G, C), jnp.float32),
        scratch_shapes=[
            pltpu.VMEM((G, H), jnp.float32),
            pltpu.VMEM((G, 1), jnp.float32),
        ],
    )(Pp, Cp, h1, batch3, wl_t, wr_t, b, wfc_t, bfc)


def kernel(x, edge_index, batch, W1l, b1, W1r, W2l, b2, W2r, Wfc, bfc):
    src = edge_index[0].reshape(NW, EPW)
    dst = edge_index[1].reshape(NW, EPW)
    zrow = jnp.zeros((N, D), jnp.float32)

    P1, C1 = _agg_counts(x, src, dst, zrow)
    h1 = _dense1(P1, C1.reshape(NC, N, 1), x,
                 W1l.T, W1r.T, b1.reshape(1, H))

    P2, _ = _agg_plain(h1, src, dst, zrow)
    out = _dense2(P2, C1.reshape(NC, N, 1), h1, batch.reshape(NB, 1, R),
                  W2l.T, W2r.T, b2.reshape(1, H),
                  Wfc.T, bfc.reshape(1, C))
    return out
